# baseline (speedup 1.0000x reference)
import functools

import jax
import jax.numpy as jnp
from jax import lax
from jax.experimental import pallas as pl
from jax.experimental.pallas import tpu as pltpu
from jax.experimental.pallas import tpu_sc as plsc

_NC, _NS = 2, 16
_NW = _NC * _NS
_L = 16
_NIN = 4
_NOUT = 2


@functools.partial(jax.jit, static_argnums=(2, 3, 4))
def _sc_pos_add(x2, pe, B, S, D):
    RPW = S // _NW
    RB = 16
    NI = RPW // RB
    NBLK = NI * B
    NCOL = D // _L

    mesh = plsc.VectorSubcoreMesh(
        core_axis_name="c", subcore_axis_name="s",
        num_cores=_NC, num_subcores=_NS)

    def body(x_hbm, pe_hbm, out_hbm, pe0, in0, in1, in2, in3, out0, out1,
             sem_p0, sem_i0, sem_i1, sem_i2, sem_i3, sem_o0, sem_o1):
        wid = lax.axis_index("s") * _NC + lax.axis_index("c")
        base = wid * RPW
        pes, sem_pe = (pe0,), (sem_p0,)
        ins, sem_in = (in0, in1, in2, in3), (sem_i0, sem_i1, sem_i2, sem_i3)
        outs, sem_out = (out0, out1), (sem_o0, sem_o1)

        def x_row(k):
            return (k % B) * S + base + (k // B) * RB

        def start_pe(i, p):
            pltpu.make_async_copy(
                pe_hbm.at[pl.ds(base + i * RB, RB)], pes[p], sem_pe[p]).start()

        def wait_pe(p):
            pltpu.make_async_copy(
                pe_hbm.at[pl.ds(0, RB)], pes[p], sem_pe[p]).wait()

        def start_in(k, j):
            pltpu.make_async_copy(
                x_hbm.at[pl.ds(x_row(k), RB)], ins[j], sem_in[j]).start()

        def wait_in(j):
            pltpu.make_async_copy(
                x_hbm.at[pl.ds(0, RB)], ins[j], sem_in[j]).wait()

        def start_out(k, j):
            pltpu.make_async_copy(
                outs[j], out_hbm.at[pl.ds(x_row(k), RB)], sem_out[j]).start()

        def wait_out(j):
            pltpu.make_async_copy(
                outs[j], out_hbm.at[pl.ds(0, RB)], sem_out[j]).wait()

        def compute(ji, jo, p):
            @plsc.parallel_loop(0, RB * NCOL, unroll=8)
            def _(t):
                r = t // NCOL
                sl = pl.ds((t % NCOL) * _L, _L)
                outs[jo][r, sl] = ins[ji][r, sl] + pes[p][r, sl]

        start_pe(0, 0)
        for j in range(_NIN):
            start_in(j, j)

        for k in range(NBLK):
            ji, jo, i, p = k % _NIN, k % _NOUT, k // B, 0
            if k % B == 0:
                wait_pe(p)
            wait_in(ji)
            if k >= _NOUT:
                wait_out(jo)
            compute(ji, jo, p)
            if k % B == B - 1 and i + 1 < NI:
                start_pe(i + 1, 0)
            start_out(k, jo)
            if k + _NIN < NBLK:
                start_in(k + _NIN, ji)
        for j in range(_NOUT):
            wait_out(j)

    return pl.kernel(
        body,
        out_type=jax.ShapeDtypeStruct((B * S, D), jnp.float32),
        mesh=mesh,
        scratch_types=[
            pltpu.VMEM((RB, D), jnp.float32),
            pltpu.VMEM((RB, D), jnp.float32),
            pltpu.VMEM((RB, D), jnp.float32),
            pltpu.VMEM((RB, D), jnp.float32),
            pltpu.VMEM((RB, D), jnp.float32),
            pltpu.VMEM((RB, D), jnp.float32),
            pltpu.VMEM((RB, D), jnp.float32),
            pltpu.SemaphoreType.DMA,
            pltpu.SemaphoreType.DMA,
            pltpu.SemaphoreType.DMA,
            pltpu.SemaphoreType.DMA,
            pltpu.SemaphoreType.DMA,
            pltpu.SemaphoreType.DMA,
            pltpu.SemaphoreType.DMA,
        ],
    )(x2, pe)


def kernel(x, pos_embedding):
    B, S, D = x.shape
    out = _sc_pos_add(x.reshape(B * S, D), pos_embedding, B, S, D)
    return out.reshape(B, S, D)


# speedup vs baseline: 1.0411x; 1.0411x over previous
import functools

import jax
import jax.numpy as jnp
from jax import lax
from jax.experimental import pallas as pl
from jax.experimental.pallas import tpu as pltpu
from jax.experimental.pallas import tpu_sc as plsc

_NC, _NS = 2, 16
_NW = _NC * _NS
_L = 16
_NIN = 2
_NOUT = 3


@functools.partial(jax.jit, static_argnums=(2, 3, 4))
def _sc_pos_add(x2, pe, B, S, D):
    RPW = S // _NW
    RB = 16
    NI = RPW // RB
    NBLK = NI * B
    NCOL = D // _L

    mesh = plsc.VectorSubcoreMesh(
        core_axis_name="c", subcore_axis_name="s",
        num_cores=_NC, num_subcores=_NS)

    def body(x_hbm, pe_hbm, out_hbm, pe0, pe1, in0, in1, out0, out1, out2,
             sem_p0, sem_p1, sem_i0, sem_i1, sem_o0, sem_o1, sem_o2):
        wid = lax.axis_index("s") * _NC + lax.axis_index("c")
        base = wid * RPW
        pes, sem_pe = (pe0, pe1), (sem_p0, sem_p1)
        ins, sem_in = (in0, in1), (sem_i0, sem_i1)
        outs, sem_out = (out0, out1, out2), (sem_o0, sem_o1, sem_o2)

        def x_row(k):
            return (k % B) * S + base + (k // B) * RB

        def start_pe(i, p):
            pltpu.make_async_copy(
                pe_hbm.at[pl.ds(base + i * RB, RB)], pes[p], sem_pe[p]).start()

        def wait_pe(p):
            pltpu.make_async_copy(
                pe_hbm.at[pl.ds(0, RB)], pes[p], sem_pe[p]).wait()

        def start_in(k, j):
            pltpu.make_async_copy(
                x_hbm.at[pl.ds(x_row(k), RB)], ins[j], sem_in[j]).start()

        def wait_in(j):
            pltpu.make_async_copy(
                x_hbm.at[pl.ds(0, RB)], ins[j], sem_in[j]).wait()

        def start_out(k, j):
            pltpu.make_async_copy(
                outs[j], out_hbm.at[pl.ds(x_row(k), RB)], sem_out[j]).start()

        def wait_out(j):
            pltpu.make_async_copy(
                outs[j], out_hbm.at[pl.ds(0, RB)], sem_out[j]).wait()

        def compute(ji, jo, p):
            @plsc.parallel_loop(0, RB * NCOL, unroll=8)
            def _(t):
                r = t // NCOL
                sl = pl.ds((t % NCOL) * _L, _L)
                outs[jo][r, sl] = ins[ji][r, sl] + pes[p][r, sl]

        start_pe(0, 0)
        start_pe(1, 1)
        for j in range(_NIN):
            start_in(j, j)

        for k in range(NBLK):
            ji, jo, i, p = k % _NIN, k % _NOUT, k // B, (k // B) % 2
            if k % B == 0:
                if 1 <= i < NI - 1:
                    start_pe(i + 1, (i + 1) % 2)
                wait_pe(p)
            wait_in(ji)
            if k >= _NOUT:
                wait_out(jo)
            compute(ji, jo, p)
            start_out(k, jo)
            if k + _NIN < NBLK:
                start_in(k + _NIN, ji)
        for j in range(_NOUT):
            wait_out(j)

    return pl.kernel(
        body,
        out_type=jax.ShapeDtypeStruct((B * S, D), jnp.float32),
        mesh=mesh,
        scratch_types=[
            pltpu.VMEM((RB, D), jnp.float32),
            pltpu.VMEM((RB, D), jnp.float32),
            pltpu.VMEM((RB, D), jnp.float32),
            pltpu.VMEM((RB, D), jnp.float32),
            pltpu.VMEM((RB, D), jnp.float32),
            pltpu.VMEM((RB, D), jnp.float32),
            pltpu.VMEM((RB, D), jnp.float32),
            pltpu.SemaphoreType.DMA,
            pltpu.SemaphoreType.DMA,
            pltpu.SemaphoreType.DMA,
            pltpu.SemaphoreType.DMA,
            pltpu.SemaphoreType.DMA,
            pltpu.SemaphoreType.DMA,
            pltpu.SemaphoreType.DMA,
        ],
    )(x2, pe)


def kernel(x, pos_embedding):
    B, S, D = x.shape
    out = _sc_pos_add(x.reshape(B * S, D), pos_embedding, B, S, D)
    return out.reshape(B, S, D)


# speedup vs baseline: 1.0910x; 1.0480x over previous
import functools

import jax
import jax.numpy as jnp
from jax import lax
from jax.experimental import pallas as pl
from jax.experimental.pallas import tpu as pltpu
from jax.experimental.pallas import tpu_sc as plsc

_NC, _NS = 2, 16
_NW = _NC * _NS
_L = 16
_NIN = 3
_NOUT = 2


@functools.partial(jax.jit, static_argnums=(2, 3, 4))
def _sc_pos_add(x2, pe, B, S, D):
    RPW = S // _NW
    RB = 16
    NI = RPW // RB
    NBLK = NI * B
    NCOL = D // _L

    mesh = plsc.VectorSubcoreMesh(
        core_axis_name="c", subcore_axis_name="s",
        num_cores=_NC, num_subcores=_NS)

    def body(x_hbm, pe_hbm, out_hbm, pe0, pe1, in0, in1, in2, out0, out1,
             sem_p0, sem_p1, sem_i0, sem_i1, sem_i2, sem_o0, sem_o1):
        wid = lax.axis_index("s") * _NC + lax.axis_index("c")
        base = wid * RPW
        pes, sem_pe = (pe0, pe1), (sem_p0, sem_p1)
        ins, sem_in = (in0, in1, in2), (sem_i0, sem_i1, sem_i2)
        outs, sem_out = (out0, out1), (sem_o0, sem_o1)

        def x_row(k):
            return (k % B) * S + base + (k // B) * RB

        def start_pe(i, p):
            pltpu.make_async_copy(
                pe_hbm.at[pl.ds(base + i * RB, RB)], pes[p], sem_pe[p]).start()

        def wait_pe(p):
            pltpu.make_async_copy(
                pe_hbm.at[pl.ds(0, RB)], pes[p], sem_pe[p]).wait()

        def start_in(k, j):
            pltpu.make_async_copy(
                x_hbm.at[pl.ds(x_row(k), RB)], ins[j], sem_in[j]).start()

        def wait_in(j):
            pltpu.make_async_copy(
                x_hbm.at[pl.ds(0, RB)], ins[j], sem_in[j]).wait()

        def start_out(k, j):
            pltpu.make_async_copy(
                outs[j], out_hbm.at[pl.ds(x_row(k), RB)], sem_out[j]).start()

        def wait_out(j):
            pltpu.make_async_copy(
                outs[j], out_hbm.at[pl.ds(0, RB)], sem_out[j]).wait()

        def compute(ji, jo, p):
            @plsc.parallel_loop(0, RB * NCOL, unroll=8)
            def _(t):
                r = t // NCOL
                sl = pl.ds((t % NCOL) * _L, _L)
                outs[jo][r, sl] = ins[ji][r, sl] + pes[p][r, sl]

        start_pe(0, 0)
        start_pe(1, 1)
        for j in range(_NIN):
            start_in(j, j)

        for k in range(NBLK):
            ji, jo, i, p = k % _NIN, k % _NOUT, k // B, (k // B) % 2
            if k % B == 0:
                if 1 <= i < NI - 1:
                    start_pe(i + 1, (i + 1) % 2)
                wait_pe(p)
            wait_in(ji)
            if k >= _NOUT:
                wait_out(jo)
            compute(ji, jo, p)
            start_out(k, jo)
            if k + _NIN < NBLK:
                start_in(k + _NIN, ji)
        for j in range(_NOUT):
            wait_out(j)

    return pl.kernel(
        body,
        out_type=jax.ShapeDtypeStruct((B * S, D), jnp.float32),
        mesh=mesh,
        scratch_types=[
            pltpu.VMEM((RB, D), jnp.float32),
            pltpu.VMEM((RB, D), jnp.float32),
            pltpu.VMEM((RB, D), jnp.float32),
            pltpu.VMEM((RB, D), jnp.float32),
            pltpu.VMEM((RB, D), jnp.float32),
            pltpu.VMEM((RB, D), jnp.float32),
            pltpu.VMEM((RB, D), jnp.float32),
            pltpu.SemaphoreType.DMA,
            pltpu.SemaphoreType.DMA,
            pltpu.SemaphoreType.DMA,
            pltpu.SemaphoreType.DMA,
            pltpu.SemaphoreType.DMA,
            pltpu.SemaphoreType.DMA,
            pltpu.SemaphoreType.DMA,
        ],
    )(x2, pe)


def kernel(x, pos_embedding):
    B, S, D = x.shape
    out = _sc_pos_add(x.reshape(B * S, D), pos_embedding, B, S, D)
    return out.reshape(B, S, D)
